# initial kernel scaffold (unmeasured)
import functools

import jax
import jax.numpy as jnp
from jax import lax
from jax.experimental import pallas as pl
from jax.experimental.pallas import tpu as pltpu

N_DEV = 32
N_STAGES = 5
N_LAYERS = 3
N_SLOTS = N_LAYERS * N_STAGES


def kernel(x, Win0, Wout0, Win1, Wout1, Win2, Wout2):
    b, d_shard = x.shape
    h_dim = Win0.shape[1]

    def body(x_ref, win0_ref, wout0_ref, win1_ref, wout1_ref, win2_ref,
             wout2_ref, out_ref, send_buf, recv_buf, send_sems, recv_sems):
        my = lax.axis_index("i")

        barrier = pltpu.get_barrier_semaphore()
        for s in range(N_STAGES):
            partner = my ^ (1 << s)
            pl.semaphore_signal(
                barrier, inc=1,
                device_id=(partner,), device_id_type=pl.DeviceIdType.MESH,
            )
        pl.semaphore_wait(barrier, N_STAGES)

        wins = [win0_ref, win1_ref, win2_ref]
        wouts = [wout0_ref, wout1_ref, wout2_ref]

        xv = x_ref[:, :].astype(jnp.bfloat16)
        for layer in range(N_LAYERS):
            w_in = wins[layer][:, :].astype(jnp.bfloat16)
            acc = jnp.dot(xv, w_in, preferred_element_type=jnp.float32)

            for s in range(N_STAGES):
                partner = my ^ (1 << s)
                slot = layer * N_STAGES + s
                send_buf[0, :, :] = acc
                rdma = pltpu.make_async_remote_copy(
                    src_ref=send_buf.at[0],
                    dst_ref=recv_buf.at[slot],
                    send_sem=send_sems.at[slot],
                    recv_sem=recv_sems.at[slot],
                    device_id=(partner,),
                    device_id_type=pl.DeviceIdType.MESH,
                )
                rdma.start()
                rdma.wait()
                acc = acc + recv_buf[slot, :, :]

            h = jnp.maximum(acc, 0.0).astype(jnp.bfloat16)
            w_out = wouts[layer][:, :].astype(jnp.bfloat16)
            if layer < N_LAYERS - 1:
                xv = jnp.dot(
                    h, w_out, preferred_element_type=jnp.float32
                ).astype(jnp.bfloat16)
            else:
                out_ref[:, :] = jnp.dot(
                    h, w_out, preferred_element_type=jnp.float32
                )

        @functools.partial(
            pl.run_scoped, second=pltpu.SemaphoreType.REGULAR
        )
        def _(second):
            for s in range(N_STAGES):
                partner = my ^ (1 << s)
                pl.semaphore_signal(
                    second, inc=1,
                    device_id=(partner,), device_id_type=pl.DeviceIdType.MESH,
                )
            pl.semaphore_wait(second, N_STAGES)

    vmem = pltpu.VMEM
    return pl.pallas_call(
        body,
        out_shape=jax.ShapeDtypeStruct((b, d_shard), jnp.float32),
        in_specs=[pl.BlockSpec(memory_space=vmem)] * 7,
        out_specs=pl.BlockSpec(memory_space=vmem),
        scratch_shapes=[
            pltpu.VMEM((1, b, h_dim), jnp.float32),
            pltpu.VMEM((N_SLOTS, b, h_dim), jnp.float32),
            pltpu.SemaphoreType.DMA((N_SLOTS,)),
            pltpu.SemaphoreType.DMA((N_SLOTS,)),
        ],
        compiler_params=pltpu.CompilerParams(collective_id=0),
    )(x, Win0, Wout0, Win1, Wout1, Win2, Wout2)


# baseline (device time: 117369 ns/iter reference)
import functools

import jax
import jax.numpy as jnp
from jax import lax
from jax.experimental import pallas as pl
from jax.experimental.pallas import tpu as pltpu

N_DEV = 32
N_STAGES = 5
N_LAYERS = 3
N_SLOTS = N_LAYERS * N_STAGES


def kernel(x, Win0, Wout0, Win1, Wout1, Win2, Wout2):
    b, d_shard = x.shape
    h_dim = Win0.shape[1]

    def body(x_ref, win0_ref, wout0_ref, win1_ref, wout1_ref, win2_ref,
             wout2_ref, out_ref, win_buf, wout_buf, copy_sems,
             send_buf, recv_buf, send_sems, recv_sems):
        my = lax.axis_index("i")

        barrier = pltpu.get_barrier_semaphore()
        for s in range(N_STAGES):
            partner = my ^ (1 << s)
            pl.semaphore_signal(
                barrier, inc=1,
                device_id=(partner,), device_id_type=pl.DeviceIdType.MESH,
            )
        pl.semaphore_wait(barrier, N_STAGES)

        wins = [win0_ref, win1_ref, win2_ref]
        wouts = [wout0_ref, wout1_ref, wout2_ref]

        xv = x_ref[:, :]
        for layer in range(N_LAYERS):
            cp_in = pltpu.make_async_copy(
                wins[layer], win_buf, copy_sems.at[0]
            )
            cp_out = pltpu.make_async_copy(
                wouts[layer], wout_buf, copy_sems.at[1]
            )
            cp_in.start()
            cp_out.start()
            cp_in.wait()
            acc = jnp.dot(
                xv, win_buf[:, :], preferred_element_type=jnp.float32
            )

            for s in range(N_STAGES):
                partner = my ^ (1 << s)
                slot = layer * N_STAGES + s
                send_buf[0, :, :] = acc.astype(jnp.bfloat16)
                rdma = pltpu.make_async_remote_copy(
                    src_ref=send_buf.at[0],
                    dst_ref=recv_buf.at[slot],
                    send_sem=send_sems.at[slot],
                    recv_sem=recv_sems.at[slot],
                    device_id=(partner,),
                    device_id_type=pl.DeviceIdType.MESH,
                )
                rdma.start()
                rdma.wait()
                acc = acc + recv_buf[slot, :, :].astype(jnp.float32)

            h = jnp.maximum(acc, 0.0)
            cp_out.wait()
            if layer < N_LAYERS - 1:
                xv = jnp.dot(
                    h, wout_buf[:, :], preferred_element_type=jnp.float32
                )
            else:
                out_ref[:, :] = jnp.dot(
                    h, wout_buf[:, :], preferred_element_type=jnp.float32
                )

        @functools.partial(
            pl.run_scoped, second=pltpu.SemaphoreType.REGULAR
        )
        def _(second):
            for s in range(N_STAGES):
                partner = my ^ (1 << s)
                pl.semaphore_signal(
                    second, inc=1,
                    device_id=(partner,), device_id_type=pl.DeviceIdType.MESH,
                )
            pl.semaphore_wait(second, N_STAGES)

    return pl.pallas_call(
        body,
        out_shape=jax.ShapeDtypeStruct((b, d_shard), jnp.float32),
        in_specs=[pl.BlockSpec(memory_space=pltpu.VMEM)]
        + [pl.BlockSpec(memory_space=pl.ANY)] * 6,
        out_specs=pl.BlockSpec(memory_space=pltpu.VMEM),
        scratch_shapes=[
            pltpu.VMEM((d_shard, h_dim), jnp.float32),
            pltpu.VMEM((h_dim, d_shard), jnp.float32),
            pltpu.SemaphoreType.DMA((2,)),
            pltpu.VMEM((1, b, h_dim), jnp.bfloat16),
            pltpu.VMEM((N_SLOTS, b, h_dim), jnp.bfloat16),
            pltpu.SemaphoreType.DMA((N_SLOTS,)),
            pltpu.SemaphoreType.DMA((N_SLOTS,)),
        ],
        compiler_params=pltpu.CompilerParams(collective_id=0),
    )(x, Win0, Wout0, Win1, Wout1, Win2, Wout2)


# device time: 89001 ns/iter; 1.3187x vs baseline; 1.3187x over previous
import functools

import jax
import jax.numpy as jnp
from jax import lax
from jax.experimental import pallas as pl
from jax.experimental.pallas import tpu as pltpu

N_DEV = 32
N_STAGES = 5
N_LAYERS = 3
N_SLOTS = N_LAYERS * N_STAGES


def kernel(x, Win0, Wout0, Win1, Wout1, Win2, Wout2, _mode="full"):
    b, d_shard = x.shape
    h_dim = Win0.shape[1]
    half = h_dim // 2

    def body(x_ref, win0_ref, wout0_ref, win1_ref, wout1_ref, win2_ref,
             wout2_ref, out_ref, win_stag, wout_stag, win_bf16, wout_bf16,
             copy_sems, send_buf, recv_a, recv_b, send_sems_a, recv_sems_a,
             send_sems_b, recv_sems_b):
        my = lax.axis_index("i")
        wins = [win0_ref, win1_ref, win2_ref]
        wouts = [wout0_ref, wout1_ref, wout2_ref]

        cp = pltpu.make_async_copy(wins[0], win_stag, copy_sems.at[0])
        cp.start()
        cp_out = pltpu.make_async_copy(wouts[0], wout_stag, copy_sems.at[1])
        cp_out.start()
        cp.wait()
        win_bf16[:, :] = win_stag[:, :].astype(jnp.bfloat16)

        barrier = pltpu.get_barrier_semaphore()
        for s in range(N_STAGES):
            partner = my ^ (1 << s)
            pl.semaphore_signal(
                barrier, inc=1,
                device_id=(partner,), device_id_type=pl.DeviceIdType.MESH,
            )
        pl.semaphore_wait(barrier, N_STAGES)

        def hidden_work(layer, s):
            if s == 0:
                pltpu.make_async_copy(
                    wouts[layer], wout_stag, copy_sems.at[1]
                ).wait()
                wout_bf16[:, :] = wout_stag[:, :].astype(jnp.bfloat16)
            elif s == 1 and layer + 1 < N_LAYERS:
                pltpu.make_async_copy(
                    wins[layer + 1], win_stag, copy_sems.at[0]
                ).start()
            elif s == 2 and layer + 1 < N_LAYERS:
                pltpu.make_async_copy(
                    wins[layer + 1], win_stag, copy_sems.at[0]
                ).wait()
                win_bf16[:, :] = win_stag[:, :].astype(jnp.bfloat16)
            elif s == 3 and layer + 1 < N_LAYERS:
                pltpu.make_async_copy(
                    wouts[layer + 1], wout_stag, copy_sems.at[1]
                ).start()

        xv = x_ref[:, :].astype(jnp.bfloat16)
        for layer in range(N_LAYERS):
            if _mode == "nomm":
                acc = jnp.zeros((b, h_dim), jnp.float32) + x_ref[0, 0]
            else:
                acc = jnp.dot(
                    xv, win_bf16[:, :], preferred_element_type=jnp.float32
                )
            acc_a = acc[:, :half]
            acc_b = acc[:, half:]

            if _mode == "noar":
                for s in range(N_STAGES):
                    hidden_work(layer, s)
            else:
                for s in range(N_STAGES):
                    slot = layer * N_STAGES + s
                    p_a = my ^ (1 << s)
                    p_b = my ^ (1 << (N_STAGES - 1 - s))
                    send_buf[0, :, :] = acc_a.astype(jnp.bfloat16)
                    send_buf[1, :, :] = acc_b.astype(jnp.bfloat16)
                    rdma_a = pltpu.make_async_remote_copy(
                        src_ref=send_buf.at[0],
                        dst_ref=recv_a.at[slot],
                        send_sem=send_sems_a.at[slot],
                        recv_sem=recv_sems_a.at[slot],
                        device_id=(p_a,),
                        device_id_type=pl.DeviceIdType.MESH,
                    )
                    rdma_b = pltpu.make_async_remote_copy(
                        src_ref=send_buf.at[1],
                        dst_ref=recv_b.at[slot],
                        send_sem=send_sems_b.at[slot],
                        recv_sem=recv_sems_b.at[slot],
                        device_id=(p_b,),
                        device_id_type=pl.DeviceIdType.MESH,
                    )
                    rdma_a.start()
                    rdma_b.start()
                    hidden_work(layer, s)
                    rdma_a.wait()
                    acc_a = acc_a + recv_a[slot, :, :].astype(jnp.float32)
                    rdma_b.wait()
                    acc_b = acc_b + recv_b[slot, :, :].astype(jnp.float32)

            h_a = jnp.maximum(acc_a, 0.0).astype(jnp.bfloat16)
            h_b = jnp.maximum(acc_b, 0.0).astype(jnp.bfloat16)
            if _mode == "nomm":
                if layer < N_LAYERS - 1:
                    xv = h_a
                else:
                    out_ref[:, :] = acc_a
            else:
                nxt = jnp.dot(
                    h_a, wout_bf16[:half, :],
                    preferred_element_type=jnp.float32,
                ) + jnp.dot(
                    h_b, wout_bf16[half:, :],
                    preferred_element_type=jnp.float32,
                )
                if layer < N_LAYERS - 1:
                    xv = nxt.astype(jnp.bfloat16)
                else:
                    out_ref[:, :] = nxt

        @functools.partial(
            pl.run_scoped, second=pltpu.SemaphoreType.REGULAR
        )
        def _(second):
            for s in range(N_STAGES):
                partner = my ^ (1 << s)
                pl.semaphore_signal(
                    second, inc=1,
                    device_id=(partner,), device_id_type=pl.DeviceIdType.MESH,
                )
            pl.semaphore_wait(second, N_STAGES)

    return pl.pallas_call(
        body,
        out_shape=jax.ShapeDtypeStruct((b, d_shard), jnp.float32),
        in_specs=[pl.BlockSpec(memory_space=pltpu.VMEM)]
        + [pl.BlockSpec(memory_space=pl.ANY)] * 6,
        out_specs=pl.BlockSpec(memory_space=pltpu.VMEM),
        scratch_shapes=[
            pltpu.VMEM((d_shard, h_dim), jnp.float32),
            pltpu.VMEM((h_dim, d_shard), jnp.float32),
            pltpu.VMEM((d_shard, h_dim), jnp.bfloat16),
            pltpu.VMEM((h_dim, d_shard), jnp.bfloat16),
            pltpu.SemaphoreType.DMA((2,)),
            pltpu.VMEM((2, b, half), jnp.bfloat16),
            pltpu.VMEM((N_SLOTS, b, half), jnp.bfloat16),
            pltpu.VMEM((N_SLOTS, b, half), jnp.bfloat16),
            pltpu.SemaphoreType.DMA((N_SLOTS,)),
            pltpu.SemaphoreType.DMA((N_SLOTS,)),
            pltpu.SemaphoreType.DMA((N_SLOTS,)),
            pltpu.SemaphoreType.DMA((N_SLOTS,)),
        ],
        compiler_params=pltpu.CompilerParams(
            collective_id=0, vmem_limit_bytes=50 * 1024 * 1024
        ),
    )(x, Win0, Wout0, Win1, Wout1, Win2, Wout2)


# device time: 80362 ns/iter; 1.4605x vs baseline; 1.1075x over previous
import functools

import jax
import jax.numpy as jnp
from jax import lax
from jax.experimental import pallas as pl
from jax.experimental.pallas import tpu as pltpu

N_DEV = 32
N_STAGES = 5
N_LAYERS = 3
N_SLOTS = N_LAYERS * N_STAGES


def kernel(x, Win0, Wout0, Win1, Wout1, Win2, Wout2, _mode="full"):
    b, d_shard = x.shape
    h_dim = Win0.shape[1]
    half = h_dim // 2

    def body(x_ref, win0_ref, wout0_ref, win1_ref, wout1_ref, win2_ref,
             wout2_ref, out_ref, win_stag, wout_stag, win_bf16, wout_bf16,
             copy_sems, send_buf, recv_a, recv_b, send_sems_a, recv_sems_a,
             send_sems_b, recv_sems_b):
        my = lax.axis_index("i")
        wins = [win0_ref, win1_ref, win2_ref]
        wouts = [wout0_ref, wout1_ref, wout2_ref]

        cp = pltpu.make_async_copy(wins[0], win_stag, copy_sems.at[0])
        cp.start()
        cp_out = pltpu.make_async_copy(wouts[0], wout_stag, copy_sems.at[1])
        cp_out.start()
        cp.wait()
        win_bf16[:, :] = win_stag[:, :].astype(jnp.bfloat16)

        barrier = pltpu.get_barrier_semaphore()
        for s in range(N_STAGES):
            partner = my ^ (1 << s)
            pl.semaphore_signal(
                barrier, inc=1,
                device_id=(partner,), device_id_type=pl.DeviceIdType.MESH,
            )
        pl.semaphore_wait(barrier, N_STAGES)

        def hidden_work(layer, s):
            if s == 0:
                pltpu.make_async_copy(
                    wouts[layer], wout_stag, copy_sems.at[1]
                ).wait()
                wout_bf16[:, :] = wout_stag[:, :].astype(jnp.bfloat16)
            elif s == 1 and layer + 1 < N_LAYERS:
                pltpu.make_async_copy(
                    wins[layer + 1], win_stag, copy_sems.at[0]
                ).start()
            elif s == 2 and layer + 1 < N_LAYERS:
                pltpu.make_async_copy(
                    wins[layer + 1], win_stag, copy_sems.at[0]
                ).wait()
                win_bf16[:, :] = win_stag[:, :].astype(jnp.bfloat16)
            elif s == 3 and layer + 1 < N_LAYERS:
                pltpu.make_async_copy(
                    wouts[layer + 1], wout_stag, copy_sems.at[1]
                ).start()

        a_bits = [1, 2, 8, 16, 4]
        b_bits = [8, 16, 2, 4, 1]

        def mk_pair(layer, s):
            slot = layer * N_STAGES + s
            rdma_a = pltpu.make_async_remote_copy(
                src_ref=send_buf.at[0],
                dst_ref=recv_a.at[slot],
                send_sem=send_sems_a.at[slot],
                recv_sem=recv_sems_a.at[slot],
                device_id=(my ^ a_bits[s],),
                device_id_type=pl.DeviceIdType.MESH,
            )
            rdma_b = pltpu.make_async_remote_copy(
                src_ref=send_buf.at[1],
                dst_ref=recv_b.at[slot],
                send_sem=send_sems_b.at[slot],
                recv_sem=recv_sems_b.at[slot],
                device_id=(my ^ b_bits[s],),
                device_id_type=pl.DeviceIdType.MESH,
            )
            return slot, rdma_a, rdma_b

        xv = x_ref[:, :].astype(jnp.bfloat16)
        for layer in range(N_LAYERS):
            if _mode != "full":
                if _mode == "nomm":
                    acc = jnp.zeros((b, h_dim), jnp.float32) + x_ref[0, 0]
                else:
                    acc = jnp.dot(
                        xv, win_bf16[:, :],
                        preferred_element_type=jnp.float32,
                    )
                acc_a = acc[:, :half]
                acc_b = acc[:, half:]
                if _mode == "noar":
                    for s in range(N_STAGES):
                        hidden_work(layer, s)
                else:
                    for s in range(N_STAGES):
                        send_buf[0, :, :] = acc_a.astype(jnp.bfloat16)
                        send_buf[1, :, :] = acc_b.astype(jnp.bfloat16)
                        slot, rdma_a, rdma_b = mk_pair(layer, s)
                        rdma_a.start()
                        rdma_b.start()
                        hidden_work(layer, s)
                        rdma_a.wait()
                        acc_a = acc_a + recv_a[slot, :, :].astype(jnp.float32)
                        rdma_b.wait()
                        acc_b = acc_b + recv_b[slot, :, :].astype(jnp.float32)
                h_a = jnp.maximum(acc_a, 0.0).astype(jnp.bfloat16)
                if layer < N_LAYERS - 1:
                    xv = h_a
                else:
                    out_ref[:, :] = acc_a
                continue

            acc_a = jnp.dot(
                xv, win_bf16[:, :half], preferred_element_type=jnp.float32
            )
            send_buf[0, :, :] = acc_a.astype(jnp.bfloat16)
            slot, rdma_a, rdma_b = mk_pair(layer, 0)
            rdma_a.start()
            acc_b = jnp.dot(
                xv, win_bf16[:, half:], preferred_element_type=jnp.float32
            )
            send_buf[1, :, :] = acc_b.astype(jnp.bfloat16)
            rdma_b.start()
            hidden_work(layer, 0)
            rdma_a.wait()
            acc_a = acc_a + recv_a[slot, :, :].astype(jnp.float32)
            rdma_b.wait()
            acc_b = acc_b + recv_b[slot, :, :].astype(jnp.float32)

            for s in range(1, N_STAGES - 1):
                send_buf[0, :, :] = acc_a.astype(jnp.bfloat16)
                send_buf[1, :, :] = acc_b.astype(jnp.bfloat16)
                slot, rdma_a, rdma_b = mk_pair(layer, s)
                rdma_a.start()
                rdma_b.start()
                hidden_work(layer, s)
                rdma_a.wait()
                acc_a = acc_a + recv_a[slot, :, :].astype(jnp.float32)
                rdma_b.wait()
                acc_b = acc_b + recv_b[slot, :, :].astype(jnp.float32)

            send_buf[0, :, :] = acc_a.astype(jnp.bfloat16)
            send_buf[1, :, :] = acc_b.astype(jnp.bfloat16)
            slot, rdma_a, rdma_b = mk_pair(layer, N_STAGES - 1)
            rdma_a.start()
            rdma_b.start()
            hidden_work(layer, N_STAGES - 1)
            rdma_a.wait()
            acc_a = acc_a + recv_a[slot, :, :].astype(jnp.float32)
            h_a = jnp.maximum(acc_a, 0.0).astype(jnp.bfloat16)
            nxt_a = jnp.dot(
                h_a, wout_bf16[:half, :], preferred_element_type=jnp.float32
            )
            rdma_b.wait()
            acc_b = acc_b + recv_b[slot, :, :].astype(jnp.float32)
            h_b = jnp.maximum(acc_b, 0.0).astype(jnp.bfloat16)
            nxt = nxt_a + jnp.dot(
                h_b, wout_bf16[half:, :], preferred_element_type=jnp.float32
            )
            if layer < N_LAYERS - 1:
                xv = nxt.astype(jnp.bfloat16)
            else:
                out_ref[:, :] = nxt

        @functools.partial(
            pl.run_scoped, second=pltpu.SemaphoreType.REGULAR
        )
        def _(second):
            for s in range(N_STAGES):
                partner = my ^ (1 << s)
                pl.semaphore_signal(
                    second, inc=1,
                    device_id=(partner,), device_id_type=pl.DeviceIdType.MESH,
                )
            pl.semaphore_wait(second, N_STAGES)

    return pl.pallas_call(
        body,
        out_shape=jax.ShapeDtypeStruct((b, d_shard), jnp.float32),
        in_specs=[pl.BlockSpec(memory_space=pltpu.VMEM)]
        + [pl.BlockSpec(memory_space=pl.ANY)] * 6,
        out_specs=pl.BlockSpec(memory_space=pltpu.VMEM),
        scratch_shapes=[
            pltpu.VMEM((d_shard, h_dim), jnp.float32),
            pltpu.VMEM((h_dim, d_shard), jnp.float32),
            pltpu.VMEM((d_shard, h_dim), jnp.bfloat16),
            pltpu.VMEM((h_dim, d_shard), jnp.bfloat16),
            pltpu.SemaphoreType.DMA((2,)),
            pltpu.VMEM((2, b, half), jnp.bfloat16),
            pltpu.VMEM((N_SLOTS, b, half), jnp.bfloat16),
            pltpu.VMEM((N_SLOTS, b, half), jnp.bfloat16),
            pltpu.SemaphoreType.DMA((N_SLOTS,)),
            pltpu.SemaphoreType.DMA((N_SLOTS,)),
            pltpu.SemaphoreType.DMA((N_SLOTS,)),
            pltpu.SemaphoreType.DMA((N_SLOTS,)),
        ],
        compiler_params=pltpu.CompilerParams(
            collective_id=0, vmem_limit_bytes=50 * 1024 * 1024
        ),
    )(x, Win0, Wout0, Win1, Wout1, Win2, Wout2)
